# baseline (speedup 1.0000x reference)
import functools
import numpy as np
import jax
import jax.numpy as jnp
from jax import lax
from jax.experimental import pallas as pl
from jax.experimental.pallas import tpu as pltpu
from jax.experimental.pallas import tpu_sc as plsc

_N = 50000
_E = 1600000
_F_IN = 395
_SEQ = 392
_C_CONV = 8
_KSZ = 5
_STRIDE = 2
_L_OUT = (_SEQ - _KSZ) // _STRIDE + 1
_FLAT = _C_CONV * _L_OUT
_D_CNN = 104
_D_HID = 40
_D_GCN = 64

_BN = 1024
_NBLK = -(-_N // _BN)
_N_UP = _NBLK * _BN
_TRASH = 2048
_ACC_TOT = 52224
_NW = 32
_EROWS = _E // 128
_EROWS_UP = 12800
_E_UP = _EROWS_UP * 128
_ROWS_PER_W = _EROWS_UP // _NW
_STEP_ROWS = 16
_NSTEPS = _ROWS_PER_W // _STEP_ROWS
_D_TAB = 8
_RT = _ACC_TOT // 16

_ci = np.arange(_C_CONV)[:, None, None]
_ki = np.arange(_KSZ)[None, :, None]
_li = np.arange(_L_OUT)[None, None, :]
_CF_ROWS = np.broadcast_to(
    3 + _STRIDE * _li + _ki, (_C_CONV, _KSZ, _L_OUT)).reshape(-1).copy()
_CF_COLS = np.broadcast_to(
    _ci * _L_OUT + _li, (_C_CONV, _KSZ, _L_OUT)).reshape(-1).copy()


def _prep_body(fcw_ref, l1w_ref, fcb_ref, l1b_ref, gwo_ref, gwr_ref, l2w_ref,
               gcb_ref, l2b_ref, w1_ref, b1t_ref, v2_ref, c0_ref):
  l1w = l1w_ref[...]
  l2w = l2w_ref[...]
  w1_ref[...] = jnp.dot(fcw_ref[...], l1w, preferred_element_type=jnp.float32)
  b1t_ref[...] = (lax.dot_general(l1w, fcb_ref[...], (((0,), (1,)), ((), ())),
                                  preferred_element_type=jnp.float32)
                  + l1b_ref[...])
  v2_ref[...] = jnp.concatenate(
      [jnp.dot(gwo_ref[...], l2w, preferred_element_type=jnp.float32),
       jnp.dot(gwr_ref[...], l2w, preferred_element_type=jnp.float32)], axis=1)
  c0_ref[...] = (jnp.dot(gcb_ref[...], l2w, preferred_element_type=jnp.float32)
                 + l2b_ref[...])


def _dense_body(xt_ref, cft_ref, bft_ref, w1_ref, b1t_ref, v2_ref,
                table_ref, st_ref):
  xt = xt_ref[...].reshape(_F_IN, _BN)
  flat_t = jnp.dot(cft_ref[...].astype(jnp.bfloat16), xt.astype(jnp.bfloat16),
                   preferred_element_type=jnp.float32)
  flat_t = jnp.maximum(flat_t + bft_ref[...], 0.0)
  h1_t = lax.dot_general(w1_ref[...], flat_t, (((0,), (0,)), ((), ())),
                         preferred_element_type=jnp.float32)
  h1_t = jnp.maximum(h1_t + b1t_ref[...], 0.0)
  st = lax.dot_general(h1_t, v2_ref[...], (((0,), (0,)), ((), ())),
                       preferred_element_type=jnp.float32)
  st_ref[...] = st
  s_col = st[:, 0:1]
  table_ref[...] = jnp.concatenate(
      [s_col, jnp.ones_like(s_col),
       jnp.zeros((s_col.shape[0], _D_TAB - 2), jnp.float32)], axis=1)


_TRT = _N_UP // 16


def _sc_scatter_body(src_hbm, dst_hbm, table_hbm, zero_hbm, acc_out,
                     srcv, dstv, valv, tab_sh, acc_sh, gsem, ssem):
  cid = lax.axis_index("c")
  sid = lax.axis_index("s")
  wid = sid * 2 + cid
  pltpu.sync_copy(zero_hbm.at[pl.ds(sid * _RT, _RT)],
                  acc_sh.at[pl.ds(sid * _RT, _RT)])
  pltpu.sync_copy(table_hbm.at[pl.ds(sid * _TRT, _TRT)],
                  tab_sh.at[pl.ds(sid * _TRT, _TRT)])
  plsc.subcore_barrier()
  base = wid * _ROWS_PER_W

  def load_and_fire(g, slot):
    r0 = base + g * _STEP_ROWS
    pltpu.sync_copy(src_hbm.at[pl.ds(r0, _STEP_ROWS)], srcv.at[slot])
    pltpu.sync_copy(dst_hbm.at[pl.ds(r0, _STEP_ROWS)], dstv.at[slot])
    for j in range(_STEP_ROWS):
      pltpu.make_async_copy(tab_sh.at[srcv.at[slot, j]], valv.at[slot, j],
                            gsem).start()

  load_and_fire(0, 0)

  def step(g, carry):
    slot = lax.rem(g, 2)
    nslot = lax.rem(g + 1, 2)
    for j in range(_STEP_ROWS):
      pltpu.make_async_copy(tab_sh.at[srcv.at[slot, j]], valv.at[slot, j],
                            gsem).wait()

    @pl.when(g + 1 < _NSTEPS)
    def _():
      load_and_fire(g + 1, nslot)

    for j in range(_STEP_ROWS):
      pltpu.make_async_copy(valv.at[slot, j], acc_sh.at[dstv.at[slot, j]],
                            ssem).start(add=True)
    for j in range(_STEP_ROWS):
      pltpu.make_async_copy(valv.at[slot, j], acc_sh.at[dstv.at[slot, j]],
                            ssem).wait()
    return carry

  lax.fori_loop(0, _NSTEPS, step, 0)
  plsc.subcore_barrier()
  pltpu.sync_copy(acc_sh.at[pl.ds(sid * _RT, _RT)],
                  acc_out.at[pl.ds(cid * _ACC_TOT + sid * _RT, _RT)])


def _combine_body(a0s_ref, a0c_ref, a1s_ref, a1c_ref, sv_ref, tv_ref, c0_ref,
                  out_ref):
  num = a0s_ref[...] + a1s_ref[...] + sv_ref[...]
  den = a0c_ref[...] + a1c_ref[...] + 1.0
  out_ref[...] = num / den + tv_ref[...] + c0_ref[0, 0]


def kernel(x, edge_index, conv_w, conv_b, fc_w, fc_b, lin1_w, lin1_b,
           gcn_w_out, gcn_w_root, gcn_b, lin2_w, lin2_b):
  f32 = jnp.float32

  w1, b1t, v2, c0 = pl.pallas_call(
      _prep_body,
      out_shape=[
          jax.ShapeDtypeStruct((_FLAT, _D_HID), f32),
          jax.ShapeDtypeStruct((_D_HID, 1), f32),
          jax.ShapeDtypeStruct((_D_HID, 2), f32),
          jax.ShapeDtypeStruct((1, 1), f32),
      ],
  )(fc_w, lin1_w, fc_b.reshape(1, _D_CNN), lin1_b.reshape(_D_HID, 1),
    gcn_w_out, gcn_w_root, lin2_w, gcn_b.reshape(1, _D_GCN),
    lin2_b.reshape(1, 1))

  cf_vals = jnp.broadcast_to(conv_w[:, 0, :, None],
                             (_C_CONV, _KSZ, _L_OUT)).reshape(-1)
  cft = jnp.zeros((_FLAT, _F_IN), f32).at[_CF_COLS, _CF_ROWS].set(cf_vals)
  bflat_t = jnp.repeat(conv_b, _L_OUT).reshape(_FLAT, 1)

  table, st = pl.pallas_call(
      _dense_body,
      grid=(_NBLK,),
      in_specs=[
          pl.BlockSpec((_F_IN, 1, _BN), lambda b: (0, 0, b)),
          pl.BlockSpec((_FLAT, _F_IN), lambda b: (0, 0)),
          pl.BlockSpec((_FLAT, 1), lambda b: (0, 0)),
          pl.BlockSpec((_FLAT, _D_HID), lambda b: (0, 0)),
          pl.BlockSpec((_D_HID, 1), lambda b: (0, 0)),
          pl.BlockSpec((_D_HID, 2), lambda b: (0, 0)),
      ],
      out_specs=[
          pl.BlockSpec((_BN, _D_TAB), lambda b: (b, 0)),
          pl.BlockSpec((_BN, 2), lambda b: (b, 0)),
      ],
      out_shape=[
          jax.ShapeDtypeStruct((_N_UP, _D_TAB), f32),
          jax.ShapeDtypeStruct((_N_UP, 2), f32),
      ],
  )(jnp.transpose(x, (2, 1, 0)), cft, bflat_t, w1, b1t, v2)

  pad = _E_UP - _E
  ar = jnp.arange(pad, dtype=jnp.int32)
  srcp = jnp.concatenate([edge_index[0], ar % _N]).reshape(_EROWS_UP, 128)
  dstp = jnp.concatenate([edge_index[1], _N_UP + (ar % _TRASH)]
                         ).reshape(_EROWS_UP, 128)
  zero = jnp.zeros((_ACC_TOT, _D_TAB), f32)

  mesh = plsc.VectorSubcoreMesh(core_axis_name="c", subcore_axis_name="s",
                                num_cores=2, num_subcores=16)
  acc = pl.kernel(
      _sc_scatter_body,
      out_type=jax.ShapeDtypeStruct((2 * _ACC_TOT, _D_TAB), f32),
      mesh=mesh,
      scratch_types=[
          pltpu.VMEM((2, _STEP_ROWS, 128), jnp.int32),
          pltpu.VMEM((2, _STEP_ROWS, 128), jnp.int32),
          pltpu.VMEM((2, _STEP_ROWS, 128, _D_TAB), f32),
          pltpu.VMEM_SHARED((_N_UP, _D_TAB), f32),
          pltpu.VMEM_SHARED((_ACC_TOT, _D_TAB), f32),
          pltpu.SemaphoreType.DMA,
          pltpu.SemaphoreType.DMA,
      ],
      compiler_params=pltpu.CompilerParams(use_tc_tiling_on_sc=False),
  )(srcp, dstp, table, zero)

  acc3 = acc.reshape(2, _ACC_TOT, _D_TAB)
  r2d = (125, 400)
  a0s = acc3[0, :_N, 0].reshape(r2d)
  a0c = acc3[0, :_N, 1].reshape(r2d)
  a1s = acc3[1, :_N, 0].reshape(r2d)
  a1c = acc3[1, :_N, 1].reshape(r2d)
  sv = st[:_N, 0].reshape(r2d)
  tv = st[:_N, 1].reshape(r2d)
  out2d = pl.pallas_call(
      _combine_body,
      in_specs=[pl.BlockSpec(r2d, lambda: (0, 0))] * 6 + [
          pl.BlockSpec(memory_space=pltpu.SMEM)],
      out_specs=pl.BlockSpec(r2d, lambda: (0, 0)),
      out_shape=jax.ShapeDtypeStruct(r2d, f32),
  )(a0s, a0c, a1s, a1c, sv, tv, c0)
  return out2d.reshape(_N)[:, None]


# speedup vs baseline: 1.1511x; 1.1511x over previous
import functools
import numpy as np
import jax
import jax.numpy as jnp
from jax import lax
from jax.experimental import pallas as pl
from jax.experimental.pallas import tpu as pltpu
from jax.experimental.pallas import tpu_sc as plsc

_N = 50000
_E = 1600000
_F_IN = 395
_SEQ = 392
_C_CONV = 8
_KSZ = 5
_STRIDE = 2
_L_OUT = (_SEQ - _KSZ) // _STRIDE + 1
_FLAT = _C_CONV * _L_OUT
_D_CNN = 104
_D_HID = 40
_D_GCN = 64

_BN = 1024
_NBLK = -(-_N // _BN)
_N_UP = _NBLK * _BN
_TRASH = 2048
_ACC_TOT = 52224
_NW = 32
_EROWS = _E // 128
_SR = 8
_TAIL0 = (_EROWS // _SR) * _SR
_TAILR = _EROWS - _TAIL0
_D_TAB = 8
_RT = _ACC_TOT // 16

_ci = np.arange(_C_CONV)[:, None, None]
_ki = np.arange(_KSZ)[None, :, None]
_li = np.arange(_L_OUT)[None, None, :]
_CF_ROWS = np.broadcast_to(
    3 + _STRIDE * _li + _ki, (_C_CONV, _KSZ, _L_OUT)).reshape(-1).copy()
_CF_COLS = np.broadcast_to(
    _ci * _L_OUT + _li, (_C_CONV, _KSZ, _L_OUT)).reshape(-1).copy()


def _prep_body(fcw_ref, l1w_ref, fcb_ref, l1b_ref, gwo_ref, gwr_ref, l2w_ref,
               gcb_ref, l2b_ref, w1_ref, b1t_ref, v2_ref, c0_ref):
  l1w = l1w_ref[...]
  l2w = l2w_ref[...]
  w1_ref[...] = jnp.dot(fcw_ref[...], l1w, preferred_element_type=jnp.float32)
  b1t_ref[...] = (lax.dot_general(l1w, fcb_ref[...], (((0,), (1,)), ((), ())),
                                  preferred_element_type=jnp.float32)
                  + l1b_ref[...])
  v2_ref[...] = jnp.concatenate(
      [jnp.dot(gwo_ref[...], l2w, preferred_element_type=jnp.float32),
       jnp.dot(gwr_ref[...], l2w, preferred_element_type=jnp.float32)], axis=1)
  c0_ref[...] = (jnp.dot(gcb_ref[...], l2w, preferred_element_type=jnp.float32)
                 + l2b_ref[...])


def _dense_body(xt_ref, cft_ref, bft_ref, w1_ref, b1t_ref, v2_ref,
                table_ref, st_ref):
  xt = xt_ref[...].reshape(_F_IN, _BN)
  flat_t = jnp.dot(cft_ref[...], xt, preferred_element_type=jnp.float32)
  flat_t = jnp.maximum(flat_t + bft_ref[...], 0.0)
  h1_t = lax.dot_general(w1_ref[...], flat_t, (((0,), (0,)), ((), ())),
                         preferred_element_type=jnp.float32)
  h1_t = jnp.maximum(h1_t + b1t_ref[...], 0.0)
  st = lax.dot_general(h1_t, v2_ref[...], (((0,), (0,)), ((), ())),
                       preferred_element_type=jnp.float32)
  st_ref[...] = st
  s_col = st[:, 0:1]
  table_ref[...] = jnp.concatenate(
      [s_col, jnp.ones_like(s_col),
       jnp.zeros((s_col.shape[0], _D_TAB - 2), jnp.float32)], axis=1)


_TRT = _N_UP // 16


def _sc_scatter_body(ei_hbm, table_hbm, zero_hbm, acc_out,
                     srcv, dstv, valv, tab_sh, acc_sh, gsem, ssem):
  cid = lax.axis_index("c")
  sid = lax.axis_index("s")
  wid = sid * 2 + cid
  pltpu.sync_copy(zero_hbm.at[pl.ds(sid * _RT, _RT)],
                  acc_sh.at[pl.ds(sid * _RT, _RT)])
  pltpu.sync_copy(table_hbm.at[pl.ds(sid * _TRT, _TRT)],
                  tab_sh.at[pl.ds(sid * _TRT, _TRT)])
  plsc.subcore_barrier()
  base = (_EROWS * wid) // _NW // _SR * _SR
  end = jnp.where(wid == _NW - 1, _TAIL0,
                  (_EROWS * (wid + 1)) // _NW // _SR * _SR)
  nsteps = (end - base) // _SR

  def load_and_fire(g, slot, rows):
    r0 = base + g * _SR
    pltpu.sync_copy(ei_hbm.at[0, pl.ds(r0, rows)],
                    srcv.at[slot, pl.ds(0, rows)])
    pltpu.sync_copy(ei_hbm.at[1, pl.ds(r0, rows)],
                    dstv.at[slot, pl.ds(0, rows)])
    for j in range(rows):
      pltpu.make_async_copy(tab_sh.at[srcv.at[slot, j]], valv.at[slot, j],
                            gsem).start()

  load_and_fire(0, 0, _SR)

  def step(g, carry):
    slot = lax.rem(g, 2)
    nslot = lax.rem(g + 1, 2)
    for j in range(_SR):
      pltpu.make_async_copy(tab_sh.at[srcv.at[slot, j]], valv.at[slot, j],
                            gsem).wait()

    @pl.when(g + 1 < nsteps)
    def _():
      load_and_fire(g + 1, nslot, _SR)

    for j in range(_SR):
      pltpu.make_async_copy(valv.at[slot, j], acc_sh.at[dstv.at[slot, j]],
                            ssem).start(add=True)
    for j in range(_SR):
      pltpu.make_async_copy(valv.at[slot, j], acc_sh.at[dstv.at[slot, j]],
                            ssem).wait()
    return carry

  lax.fori_loop(0, nsteps, step, 0)

  @pl.when(wid == _NW - 1)
  def _():
    pltpu.sync_copy(ei_hbm.at[0, pl.ds(_TAIL0, _TAILR)],
                    srcv.at[0, pl.ds(0, _TAILR)])
    pltpu.sync_copy(ei_hbm.at[1, pl.ds(_TAIL0, _TAILR)],
                    dstv.at[0, pl.ds(0, _TAILR)])
    for j in range(_TAILR):
      pltpu.make_async_copy(tab_sh.at[srcv.at[0, j]], valv.at[0, j],
                            gsem).start()
    for j in range(_TAILR):
      pltpu.make_async_copy(tab_sh.at[srcv.at[0, j]], valv.at[0, j],
                            gsem).wait()
    for j in range(_TAILR):
      pltpu.make_async_copy(valv.at[0, j], acc_sh.at[dstv.at[0, j]],
                            ssem).start(add=True)
    for j in range(_TAILR):
      pltpu.make_async_copy(valv.at[0, j], acc_sh.at[dstv.at[0, j]],
                            ssem).wait()

  plsc.subcore_barrier()
  pltpu.sync_copy(acc_sh.at[pl.ds(sid * _RT, _RT)],
                  acc_out.at[pl.ds(cid * _ACC_TOT + sid * _RT, _RT)])


def _combine_body(a0s_ref, a0c_ref, a1s_ref, a1c_ref, sv_ref, tv_ref, c0_ref,
                  out_ref):
  num = a0s_ref[...] + a1s_ref[...] + sv_ref[...]
  den = a0c_ref[...] + a1c_ref[...] + 1.0
  out_ref[...] = num / den + tv_ref[...] + c0_ref[0, 0]


def kernel(x, edge_index, conv_w, conv_b, fc_w, fc_b, lin1_w, lin1_b,
           gcn_w_out, gcn_w_root, gcn_b, lin2_w, lin2_b):
  f32 = jnp.float32

  w1, b1t, v2, c0 = pl.pallas_call(
      _prep_body,
      out_shape=[
          jax.ShapeDtypeStruct((_FLAT, _D_HID), f32),
          jax.ShapeDtypeStruct((_D_HID, 1), f32),
          jax.ShapeDtypeStruct((_D_HID, 2), f32),
          jax.ShapeDtypeStruct((1, 1), f32),
      ],
  )(fc_w, lin1_w, fc_b.reshape(1, _D_CNN), lin1_b.reshape(_D_HID, 1),
    gcn_w_out, gcn_w_root, lin2_w, gcn_b.reshape(1, _D_GCN),
    lin2_b.reshape(1, 1))

  cf_vals = jnp.broadcast_to(conv_w[:, 0, :, None],
                             (_C_CONV, _KSZ, _L_OUT)).reshape(-1)
  cft = jnp.zeros((_FLAT, _F_IN), f32).at[_CF_COLS, _CF_ROWS].set(cf_vals)
  bflat_t = jnp.repeat(conv_b, _L_OUT).reshape(_FLAT, 1)

  table, st = pl.pallas_call(
      _dense_body,
      grid=(_NBLK,),
      in_specs=[
          pl.BlockSpec((_F_IN, 1, _BN), lambda b: (0, 0, b)),
          pl.BlockSpec((_FLAT, _F_IN), lambda b: (0, 0)),
          pl.BlockSpec((_FLAT, 1), lambda b: (0, 0)),
          pl.BlockSpec((_FLAT, _D_HID), lambda b: (0, 0)),
          pl.BlockSpec((_D_HID, 1), lambda b: (0, 0)),
          pl.BlockSpec((_D_HID, 2), lambda b: (0, 0)),
      ],
      out_specs=[
          pl.BlockSpec((_BN, _D_TAB), lambda b: (b, 0)),
          pl.BlockSpec((_BN, 2), lambda b: (b, 0)),
      ],
      out_shape=[
          jax.ShapeDtypeStruct((_N_UP, _D_TAB), f32),
          jax.ShapeDtypeStruct((_N_UP, 2), f32),
      ],
  )(jnp.transpose(x, (2, 1, 0)), cft, bflat_t, w1, b1t, v2)

  ei3 = edge_index.reshape(2, _EROWS, 128)
  zero = jnp.zeros((_ACC_TOT, _D_TAB), f32)

  mesh = plsc.VectorSubcoreMesh(core_axis_name="c", subcore_axis_name="s",
                                num_cores=2, num_subcores=16)
  acc = pl.kernel(
      _sc_scatter_body,
      out_type=jax.ShapeDtypeStruct((2 * _ACC_TOT, _D_TAB), f32),
      mesh=mesh,
      scratch_types=[
          pltpu.VMEM((2, _SR, 128), jnp.int32),
          pltpu.VMEM((2, _SR, 128), jnp.int32),
          pltpu.VMEM((2, _SR, 128, _D_TAB), f32),
          pltpu.VMEM_SHARED((_N_UP, _D_TAB), f32),
          pltpu.VMEM_SHARED((_ACC_TOT, _D_TAB), f32),
          pltpu.SemaphoreType.DMA,
          pltpu.SemaphoreType.DMA,
      ],
      compiler_params=pltpu.CompilerParams(use_tc_tiling_on_sc=False),
  )(ei3, table, zero)

  acc3 = acc.reshape(2, _ACC_TOT, _D_TAB)
  r2d = (125, 400)
  a0s = acc3[0, :_N, 0].reshape(r2d)
  a0c = acc3[0, :_N, 1].reshape(r2d)
  a1s = acc3[1, :_N, 0].reshape(r2d)
  a1c = acc3[1, :_N, 1].reshape(r2d)
  sv = st[:_N, 0].reshape(r2d)
  tv = st[:_N, 1].reshape(r2d)
  out2d = pl.pallas_call(
      _combine_body,
      in_specs=[pl.BlockSpec(r2d, lambda: (0, 0))] * 6 + [
          pl.BlockSpec(memory_space=pltpu.SMEM)],
      out_specs=pl.BlockSpec(r2d, lambda: (0, 0)),
      out_shape=jax.ShapeDtypeStruct(r2d, f32),
  )(a0s, a0c, a1s, a1c, sv, tv, c0)
  return out2d.reshape(_N)[:, None]


# speedup vs baseline: 1.1788x; 1.0241x over previous
import functools
import numpy as np
import jax
import jax.numpy as jnp
from jax import lax
from jax.experimental import pallas as pl
from jax.experimental.pallas import tpu as pltpu
from jax.experimental.pallas import tpu_sc as plsc

_N = 50000
_E = 1600000
_F_IN = 395
_SEQ = 392
_C_CONV = 8
_KSZ = 5
_STRIDE = 2
_L_OUT = (_SEQ - _KSZ) // _STRIDE + 1
_FLAT = _C_CONV * _L_OUT
_D_CNN = 104
_D_HID = 40
_D_GCN = 64

_BN = 2048
_NBLK = -(-_N // _BN)
_N_UP = _NBLK * _BN
_TRASH = 2048
_ACC_TOT = 52224
_NW = 32
_EROWS = _E // 128
_SR = 8
_TAIL0 = (_EROWS // _SR) * _SR
_TAILR = _EROWS - _TAIL0
_D_TAB = 8
_RT = _ACC_TOT // 16

_ci = np.arange(_C_CONV)[:, None, None]
_ki = np.arange(_KSZ)[None, :, None]
_li = np.arange(_L_OUT)[None, None, :]
_CF_ROWS = np.broadcast_to(
    3 + _STRIDE * _li + _ki, (_C_CONV, _KSZ, _L_OUT)).reshape(-1).copy()
_CF_COLS = np.broadcast_to(
    _ci * _L_OUT + _li, (_C_CONV, _KSZ, _L_OUT)).reshape(-1).copy()


def _prep_body(fcw_ref, l1w_ref, fcb_ref, l1b_ref, gwo_ref, gwr_ref, l2w_ref,
               gcb_ref, l2b_ref, w1_ref, b1t_ref, v2_ref, c0_ref):
  l1w = l1w_ref[...]
  l2w = l2w_ref[...]
  w1_ref[...] = jnp.dot(fcw_ref[...], l1w, preferred_element_type=jnp.float32)
  b1t_ref[...] = (lax.dot_general(l1w, fcb_ref[...], (((0,), (1,)), ((), ())),
                                  preferred_element_type=jnp.float32)
                  + l1b_ref[...])
  v2_ref[...] = jnp.concatenate(
      [jnp.dot(gwo_ref[...], l2w, preferred_element_type=jnp.float32),
       jnp.dot(gwr_ref[...], l2w, preferred_element_type=jnp.float32)], axis=1)
  c0_ref[...] = (jnp.dot(gcb_ref[...], l2w, preferred_element_type=jnp.float32)
                 + l2b_ref[...])


def _dense_body(xt_ref, cft_ref, bft_ref, w1_ref, b1t_ref, v2_ref,
                table_ref, st_ref):
  xt = xt_ref[...].reshape(_F_IN, _BN)
  flat_t = jnp.dot(cft_ref[...], xt, preferred_element_type=jnp.float32)
  flat_t = jnp.maximum(flat_t + bft_ref[...], 0.0)
  h1_t = lax.dot_general(w1_ref[...], flat_t, (((0,), (0,)), ((), ())),
                         preferred_element_type=jnp.float32)
  h1_t = jnp.maximum(h1_t + b1t_ref[...], 0.0)
  st = lax.dot_general(h1_t, v2_ref[...], (((0,), (0,)), ((), ())),
                       preferred_element_type=jnp.float32)
  st_ref[...] = st
  s_col = st[:, 0:1]
  table_ref[...] = jnp.concatenate(
      [s_col, jnp.ones_like(s_col),
       jnp.zeros((s_col.shape[0], _D_TAB - 2), jnp.float32)], axis=1)


_TRT = _N_UP // 16


def _sc_scatter_body(ei_hbm, table_hbm, zero_hbm, acc_out,
                     srcv, dstv, valv, tab_sh, acc_sh, gsem, ssem):
  cid = lax.axis_index("c")
  sid = lax.axis_index("s")
  wid = sid * 2 + cid
  pltpu.sync_copy(zero_hbm.at[pl.ds(sid * _RT, _RT)],
                  acc_sh.at[pl.ds(sid * _RT, _RT)])
  pltpu.sync_copy(table_hbm.at[pl.ds(sid * _TRT, _TRT)],
                  tab_sh.at[pl.ds(sid * _TRT, _TRT)])
  plsc.subcore_barrier()
  base = (_EROWS * wid) // _NW // _SR * _SR
  end = jnp.where(wid == _NW - 1, _TAIL0,
                  (_EROWS * (wid + 1)) // _NW // _SR * _SR)
  nsteps = (end - base) // _SR

  def load_and_fire(g, slot, rows):
    r0 = base + g * _SR
    pltpu.sync_copy(ei_hbm.at[0, pl.ds(r0, rows)],
                    srcv.at[slot, pl.ds(0, rows)])
    pltpu.sync_copy(ei_hbm.at[1, pl.ds(r0, rows)],
                    dstv.at[slot, pl.ds(0, rows)])
    for j in range(rows):
      pltpu.make_async_copy(tab_sh.at[srcv.at[slot, j]], valv.at[slot, j],
                            gsem).start()

  load_and_fire(0, 0, _SR)

  def step(g, carry):
    slot = lax.rem(g, 2)
    nslot = lax.rem(g + 1, 2)
    for j in range(_SR):
      pltpu.make_async_copy(tab_sh.at[srcv.at[slot, j]], valv.at[slot, j],
                            gsem).wait()

    @pl.when(g + 1 < nsteps)
    def _():
      load_and_fire(g + 1, nslot, _SR)

    for j in range(_SR):
      pltpu.make_async_copy(valv.at[slot, j], acc_sh.at[dstv.at[slot, j]],
                            ssem).start(add=True)
    for j in range(_SR):
      pltpu.make_async_copy(valv.at[slot, j], acc_sh.at[dstv.at[slot, j]],
                            ssem).wait()
    return carry

  lax.fori_loop(0, nsteps, step, 0)

  @pl.when(wid == _NW - 1)
  def _():
    pltpu.sync_copy(ei_hbm.at[0, pl.ds(_TAIL0, _TAILR)],
                    srcv.at[0, pl.ds(0, _TAILR)])
    pltpu.sync_copy(ei_hbm.at[1, pl.ds(_TAIL0, _TAILR)],
                    dstv.at[0, pl.ds(0, _TAILR)])
    for j in range(_TAILR):
      pltpu.make_async_copy(tab_sh.at[srcv.at[0, j]], valv.at[0, j],
                            gsem).start()
    for j in range(_TAILR):
      pltpu.make_async_copy(tab_sh.at[srcv.at[0, j]], valv.at[0, j],
                            gsem).wait()
    for j in range(_TAILR):
      pltpu.make_async_copy(valv.at[0, j], acc_sh.at[dstv.at[0, j]],
                            ssem).start(add=True)
    for j in range(_TAILR):
      pltpu.make_async_copy(valv.at[0, j], acc_sh.at[dstv.at[0, j]],
                            ssem).wait()

  plsc.subcore_barrier()
  pltpu.sync_copy(acc_sh.at[pl.ds(sid * _RT, _RT)],
                  acc_out.at[pl.ds(cid * _ACC_TOT + sid * _RT, _RT)])


def _combine_body(a0s_ref, a0c_ref, a1s_ref, a1c_ref, sv_ref, tv_ref, c0_ref,
                  out_ref):
  num = a0s_ref[...] + a1s_ref[...] + sv_ref[...]
  den = a0c_ref[...] + a1c_ref[...] + 1.0
  out_ref[...] = num / den + tv_ref[...] + c0_ref[0, 0]


def kernel(x, edge_index, conv_w, conv_b, fc_w, fc_b, lin1_w, lin1_b,
           gcn_w_out, gcn_w_root, gcn_b, lin2_w, lin2_b):
  f32 = jnp.float32

  w1, b1t, v2, c0 = pl.pallas_call(
      _prep_body,
      out_shape=[
          jax.ShapeDtypeStruct((_FLAT, _D_HID), f32),
          jax.ShapeDtypeStruct((_D_HID, 1), f32),
          jax.ShapeDtypeStruct((_D_HID, 2), f32),
          jax.ShapeDtypeStruct((1, 1), f32),
      ],
  )(fc_w, lin1_w, fc_b.reshape(1, _D_CNN), lin1_b.reshape(_D_HID, 1),
    gcn_w_out, gcn_w_root, lin2_w, gcn_b.reshape(1, _D_GCN),
    lin2_b.reshape(1, 1))

  cf_vals = jnp.broadcast_to(conv_w[:, 0, :, None],
                             (_C_CONV, _KSZ, _L_OUT)).reshape(-1)
  cft = jnp.zeros((_FLAT, _F_IN), f32).at[_CF_COLS, _CF_ROWS].set(cf_vals)
  bflat_t = jnp.repeat(conv_b, _L_OUT).reshape(_FLAT, 1)

  table, st = pl.pallas_call(
      _dense_body,
      grid=(_NBLK,),
      in_specs=[
          pl.BlockSpec((_F_IN, 1, _BN), lambda b: (0, 0, b)),
          pl.BlockSpec((_FLAT, _F_IN), lambda b: (0, 0)),
          pl.BlockSpec((_FLAT, 1), lambda b: (0, 0)),
          pl.BlockSpec((_FLAT, _D_HID), lambda b: (0, 0)),
          pl.BlockSpec((_D_HID, 1), lambda b: (0, 0)),
          pl.BlockSpec((_D_HID, 2), lambda b: (0, 0)),
      ],
      out_specs=[
          pl.BlockSpec((_BN, _D_TAB), lambda b: (b, 0)),
          pl.BlockSpec((_BN, 2), lambda b: (b, 0)),
      ],
      out_shape=[
          jax.ShapeDtypeStruct((_N_UP, _D_TAB), f32),
          jax.ShapeDtypeStruct((_N_UP, 2), f32),
      ],
  )(jnp.transpose(x, (2, 1, 0)), cft, bflat_t, w1, b1t, v2)

  ei3 = edge_index.reshape(2, _EROWS, 128)
  zero = jnp.zeros((_ACC_TOT, _D_TAB), f32)

  mesh = plsc.VectorSubcoreMesh(core_axis_name="c", subcore_axis_name="s",
                                num_cores=2, num_subcores=16)
  acc = pl.kernel(
      _sc_scatter_body,
      out_type=jax.ShapeDtypeStruct((2 * _ACC_TOT, _D_TAB), f32),
      mesh=mesh,
      scratch_types=[
          pltpu.VMEM((2, _SR, 128), jnp.int32),
          pltpu.VMEM((2, _SR, 128), jnp.int32),
          pltpu.VMEM((2, _SR, 128, _D_TAB), f32),
          pltpu.VMEM_SHARED((_N_UP, _D_TAB), f32),
          pltpu.VMEM_SHARED((_ACC_TOT, _D_TAB), f32),
          pltpu.SemaphoreType.DMA,
          pltpu.SemaphoreType.DMA,
      ],
      compiler_params=pltpu.CompilerParams(use_tc_tiling_on_sc=False),
  )(ei3, table, zero)

  acc3 = acc.reshape(2, _ACC_TOT, _D_TAB)
  r2d = (125, 400)
  a0s = acc3[0, :_N, 0].reshape(r2d)
  a0c = acc3[0, :_N, 1].reshape(r2d)
  a1s = acc3[1, :_N, 0].reshape(r2d)
  a1c = acc3[1, :_N, 1].reshape(r2d)
  sv = st[:_N, 0].reshape(r2d)
  tv = st[:_N, 1].reshape(r2d)
  out2d = pl.pallas_call(
      _combine_body,
      in_specs=[pl.BlockSpec(r2d, lambda: (0, 0))] * 6 + [
          pl.BlockSpec(memory_space=pltpu.SMEM)],
      out_specs=pl.BlockSpec(r2d, lambda: (0, 0)),
      out_shape=jax.ShapeDtypeStruct(r2d, f32),
  )(a0s, a0c, a1s, a1c, sv, tv, c0)
  return out2d.reshape(_N)[:, None]
